# Initial kernel scaffold; baseline (speedup 1.0000x reference)
import jax
import jax.numpy as jnp
from jax.experimental import pallas as pl


def kernel(x, edge_index, W_self1, W_nbr1, b1, W_self2, W_nbr2, b2, P_w1, P_b1, P_w2, P_b2):
    raise NotImplementedError("write your pallas kernel here")


# speedup vs baseline: 4.1826x; 4.1826x over previous
import functools

import jax
import jax.numpy as jnp
import numpy as np
from jax import lax
from jax.experimental import pallas as pl
from jax.experimental.pallas import tpu as pltpu
from jax.experimental.pallas import tpu_sc as plsc

N = 10000
E = 160000
D = 256
H = 128
TARGET_PCT = 0.1

NS = 16
EPS = E // NS
CH = 128
NCHUNK = -(-EPS // CH)
EP = NCHUNK * CH
NP = 10112
RPS = NP // NS
DRS = 632
DRS_LAST = N - (NS - 1) * DRS

BR = 400
NB = N // BR


def _stats_body(x_ref, s_ref, q_ref):
    i = pl.program_id(0)

    @pl.when(i == 0)
    def _():
        s_ref[...] = jnp.zeros_like(s_ref)
        q_ref[...] = jnp.zeros_like(q_ref)

    xb = x_ref[...]
    s_ref[...] += jnp.sum(xb, axis=0, keepdims=True)
    q_ref[...] += jnp.sum(xb * xb, axis=0, keepdims=True)


def _stats(x):
    return pl.pallas_call(
        _stats_body,
        grid=(NB,),
        in_specs=[pl.BlockSpec((BR, D), lambda i: (i, 0))],
        out_specs=[pl.BlockSpec((1, D), lambda i: (0, 0)),
                   pl.BlockSpec((1, D), lambda i: (0, 0))],
        out_shape=[jax.ShapeDtypeStruct((1, D), jnp.float32),
                   jax.ShapeDtypeStruct((1, D), jnp.float32)],
    )(x)


def _mask_body(x_ref, m_ref, z_ref, s_ref, q_ref, o_ref):
    mean = jnp.sum(s_ref[...]) / (N * D)
    var = jnp.sum(q_ref[...]) / (N * D) - mean * mean
    std = jnp.sqrt(jnp.maximum(var, 0.0))
    tok = z_ref[...] * std + mean
    m = m_ref[...]
    o_ref[...] = x_ref[...] * (1.0 - m) + m * tok


def _mask(x, mask, z, s, q):
    return pl.pallas_call(
        _mask_body,
        grid=(2, NB),
        in_specs=[
            pl.BlockSpec((BR, H), lambda j, i: (i, j)),
            pl.BlockSpec((BR, 1), lambda j, i: (i, 0)),
            pl.BlockSpec((1, H), lambda j, i: (0, j)),
            pl.BlockSpec((1, D), lambda j, i: (0, 0)),
            pl.BlockSpec((1, D), lambda j, i: (0, 0)),
        ],
        out_specs=pl.BlockSpec((BR, H), lambda j, i: (j * NB + i, 0)),
        out_shape=jax.ShapeDtypeStruct((2 * N, H), jnp.float32),
    )(x, mask, z, s, q)


def _conv_body(ht_ref, hb_ref, at_ref, ab_ref, dg_ref, ws_ref, wn_ref, b_ref,
               o_ref):
    invd = 1.0 / jnp.maximum(dg_ref[...], 1.0)
    at = at_ref[...] * invd
    ab = ab_ref[...] * invd
    ws = ws_ref[...]
    wn = wn_ref[...]
    acc = jnp.dot(ht_ref[...], ws[:H], preferred_element_type=jnp.float32)
    acc += jnp.dot(hb_ref[...], ws[H:], preferred_element_type=jnp.float32)
    acc += jnp.dot(at, wn[:H], preferred_element_type=jnp.float32)
    acc += jnp.dot(ab, wn[H:], preferred_element_type=jnp.float32)
    o_ref[...] = jnp.maximum(acc + b_ref[...], 0.0)


_SPEC_T = pl.BlockSpec((BR, H), lambda j, i: (i, 0))
_SPEC_B = pl.BlockSpec((BR, H), lambda j, i: (i + NB, 0))
_SPEC_DG = pl.BlockSpec((BR, 1), lambda j, i: (i, 0))
_SPEC_W = pl.BlockSpec((D, H), lambda j, i: (0, j))
_SPEC_BIAS = pl.BlockSpec((1, H), lambda j, i: (0, j))
_SPEC_OUT = pl.BlockSpec((BR, H), lambda j, i: (j * NB + i, 0))
_STACKED = jax.ShapeDtypeStruct((2 * N, H), jnp.float32)


def _conv(h, a, deg, ws, wn, b):
    return pl.pallas_call(
        _conv_body,
        grid=(2, NB),
        in_specs=[_SPEC_T, _SPEC_B, _SPEC_T, _SPEC_B, _SPEC_DG, _SPEC_W,
                  _SPEC_W, _SPEC_BIAS],
        out_specs=_SPEC_OUT,
        out_shape=_STACKED,
    )(h, h, a, a, deg, ws, wn, b)


def _conv_jepa_body(ht_ref, hb_ref, at_ref, ab_ref, dg_ref, ws_ref, wn_ref,
                    b_ref, pw_ref, pb_ref, o_ref, p_ref):
    invd = 1.0 / jnp.maximum(dg_ref[...], 1.0)
    at = at_ref[...] * invd
    ab = ab_ref[...] * invd
    ws = ws_ref[...]
    wn = wn_ref[...]
    pw = pw_ref[...]
    acc = jnp.dot(ht_ref[...], ws[:H], preferred_element_type=jnp.float32)
    acc += jnp.dot(hb_ref[...], ws[H:], preferred_element_type=jnp.float32)
    acc += jnp.dot(at, wn[:H], preferred_element_type=jnp.float32)
    acc += jnp.dot(ab, wn[H:], preferred_element_type=jnp.float32)
    o_ref[...] = jnp.maximum(acc + b_ref[...], 0.0)
    p = jnp.dot(at, pw[:H], preferred_element_type=jnp.float32)
    p += jnp.dot(ab, pw[H:], preferred_element_type=jnp.float32)
    p_ref[...] = p + pb_ref[...]


def _conv_jepa(h, a, deg, ws, wn, b, pw, pb):
    return pl.pallas_call(
        _conv_jepa_body,
        grid=(2, NB),
        in_specs=[_SPEC_T, _SPEC_B, _SPEC_T, _SPEC_B, _SPEC_DG, _SPEC_W,
                  _SPEC_W, _SPEC_BIAS, _SPEC_W, _SPEC_BIAS],
        out_specs=[_SPEC_OUT, _SPEC_OUT],
        out_shape=[_STACKED, _STACKED],
    )(h, h, a, a, deg, ws, wn, b, pw, pb)


def _jepa_body(at_ref, ab_ref, dg_ref, pw_ref, pb_ref, o_ref):
    invd = 1.0 / jnp.maximum(dg_ref[...], 1.0)
    at = at_ref[...] * invd
    ab = ab_ref[...] * invd
    pw = pw_ref[...]
    p = jnp.dot(at, pw[:H], preferred_element_type=jnp.float32)
    p += jnp.dot(ab, pw[H:], preferred_element_type=jnp.float32)
    o_ref[...] = p + pb_ref[...]


def _jepa(a, deg, pw, pb):
    return pl.pallas_call(
        _jepa_body,
        grid=(2, NB),
        in_specs=[_SPEC_T, _SPEC_B, _SPEC_DG, _SPEC_W, _SPEC_BIAS],
        out_specs=_SPEC_OUT,
        out_shape=_STACKED,
    )(a, a, deg, pw, pb)


@functools.lru_cache(maxsize=None)
def _make_sc_agg(compute_deg):
    mesh = plsc.VectorSubcoreMesh(core_axis_name="c", subcore_axis_name="s",
                                  num_cores=2, num_subcores=NS)
    out_type = [jax.ShapeDtypeStruct((2 * N, H), jnp.float32)]
    if compute_deg:
        out_type.append(jax.ShapeDtypeStruct((N,), jnp.float32))
    scratch = [
        pltpu.VMEM((NCHUNK, CH), jnp.int32),
        pltpu.VMEM((NCHUNK, CH), jnp.int32),
        pltpu.VMEM((CH, H), jnp.float32),
        pltpu.VMEM((CH,), jnp.float32),
        pltpu.VMEM((CH,), jnp.float32),
        pltpu.VMEM((DRS,), jnp.float32),
        pltpu.VMEM_SHARED((NP, H), jnp.float32),
        pltpu.VMEM_SHARED((NP,), jnp.float32),
        pltpu.SemaphoreType.DMA,
    ]

    def body(h_hbm, srcp_hbm, dstp_hbm, out_hbm, *rest):
        if compute_deg:
            deg_hbm = rest[0]
            rest = rest[1:]
        src_v, dst_v, rows_v, ones_v, zero_v, deg_buf, acc_sh, deg_sh, \
            sem = rest

        c = lax.axis_index("c")
        s = lax.axis_index("s")

        pltpu.sync_copy(srcp_hbm.at[c, s], src_v)
        pltpu.sync_copy(dstp_hbm.at[s], dst_v)

        z16 = jnp.zeros((16,), jnp.float32)

        def zrow(i, carry):
            def zcol(k, carry2):
                rows_v[i, pl.ds(k * 16, 16)] = z16
                return carry2
            return lax.fori_loop(0, H // 16, zcol, carry)

        lax.fori_loop(0, CH, zrow, 0)

        def ocol(k, carry):
            ones_v[pl.ds(k * 16, 16)] = z16 + 1.0
            zero_v[pl.ds(k * 16, 16)] = z16
            return carry

        lax.fori_loop(0, CH // 16, ocol, 0)

        zb = s * RPS
        for k in range(4):
            pltpu.sync_copy(rows_v, acc_sh.at[pl.ds(zb + k * CH, CH)])
        pltpu.sync_copy(rows_v.at[pl.ds(0, RPS - 4 * CH)],
                        acc_sh.at[pl.ds(zb + 4 * CH, RPS - 4 * CH)])
        if compute_deg:
            @pl.when(c == 0)
            def _():
                for k in range(4):
                    pltpu.sync_copy(zero_v, deg_sh.at[pl.ds(zb + k * CH, CH)])
                pltpu.sync_copy(zero_v.at[pl.ds(0, RPS - 4 * CH)],
                                deg_sh.at[pl.ds(zb + 4 * CH, RPS - 4 * CH)])
        plsc.subcore_barrier()

        def step(j, carry):
            pltpu.async_copy(h_hbm.at[src_v.at[j]], rows_v, sem).wait()
            pltpu.sync_copy(rows_v, acc_sh.at[dst_v.at[j]], add=True)
            if compute_deg:
                @pl.when(c == 0)
                def _():
                    pltpu.sync_copy(ones_v, deg_sh.at[dst_v.at[j]], add=True)
            return carry

        lax.fori_loop(0, NCHUNK, step, 0)
        plsc.subcore_barrier()

        base = s * DRS

        @pl.when(s < NS - 1)
        def _():
            pltpu.sync_copy(acc_sh.at[pl.ds(base, DRS)],
                            out_hbm.at[pl.ds(c * N + base, DRS)])

        @pl.when(s == NS - 1)
        def _():
            pltpu.sync_copy(acc_sh.at[pl.ds((NS - 1) * DRS, DRS_LAST)],
                            out_hbm.at[pl.ds(c * N + (NS - 1) * DRS,
                                             DRS_LAST)])

        if compute_deg:
            @pl.when((c == 0) & (s < NS - 1))
            def _():
                pltpu.sync_copy(deg_sh.at[pl.ds(base, DRS)], deg_buf)
                pltpu.sync_copy(deg_buf, deg_hbm.at[pl.ds(base, DRS)])

            @pl.when((c == 0) & (s == NS - 1))
            def _():
                pltpu.sync_copy(deg_sh.at[pl.ds((NS - 1) * DRS, DRS_LAST)],
                                deg_buf.at[pl.ds(0, DRS_LAST)])
                pltpu.sync_copy(deg_buf.at[pl.ds(0, DRS_LAST)],
                                deg_hbm.at[pl.ds((NS - 1) * DRS, DRS_LAST)])

    return pl.kernel(
        body,
        out_type=tuple(out_type) if compute_deg else out_type[0],
        mesh=mesh,
        scratch_types=scratch,
    )


def _sc_agg_deg(*args):
    return _make_sc_agg(True)(*args)


def _sc_agg(*args):
    return _make_sc_agg(False)(*args)


def kernel(x, edge_index, W_self1, W_nbr1, b1, W_self2, W_nbr2, b2,
           P_w1, P_b1, P_w2, P_b2):
    src = edge_index[0]
    dst = edge_index[1]

    key = jax.random.key(42)
    k1, k2 = jax.random.split(key)
    n_tgt = int(TARGET_PCT * N)
    tgt_idx = jax.random.permutation(k1, N)[:n_tgt]
    mask = jnp.zeros((N, 1), dtype=x.dtype).at[tgt_idx].set(1.0)
    z = jax.random.normal(k2, (1, D), dtype=x.dtype)

    srcs = jnp.pad(src.reshape(NS, EPS), ((0, 0), (0, EP - EPS)))
    srcp = (srcs[None] + jnp.array([0, N], jnp.int32)[:, None, None])
    srcp = srcp.reshape(2, NS, NCHUNK, CH)
    dstp = jnp.pad(dst.reshape(NS, EPS), ((0, 0), (0, EP - EPS)),
                   constant_values=N).reshape(NS, NCHUNK, CH)

    ssum, ssq = _stats(x)
    xm = _mask(x, mask, z, ssum, ssq)

    a0, deg = _sc_agg_deg(xm, srcp, dstp)
    deg = deg[:, None]
    x1 = _conv(xm, a0, deg, W_self1, W_nbr1, b1[None, :])
    a1 = _sc_agg(x1, srcp, dstp)
    x2, h1 = _conv_jepa(x1, a1, deg, W_self2, W_nbr2, b2[None, :],
                        P_w1, P_b1[None, :])
    a2 = _sc_agg(x2, srcp, dstp)
    h2 = _jepa(a2, deg, P_w2, P_b2[None, :])

    unstack = lambda t: jnp.concatenate([t[:N], t[N:]], axis=1)
    return ((unstack(h1), unstack(h2)), tgt_idx, unstack(x2))
